# baseline (speedup 1.0000x reference)
import functools

import jax
import jax.numpy as jnp
from jax import lax
from jax.experimental import pallas as pl
from jax.experimental.pallas import tpu as pltpu
from jax.experimental.pallas import tpu_sc as plsc

DEPTH = 1000
BATCH = 16384

_info = plsc.get_sparse_core_info()
_NC, _NS, _L = _info.num_cores, _info.num_subcores, _info.num_lanes
_NW = _NC * _NS
_COLS_PER_W = BATCH // _NW
_CHUNK_COLS = 256
_N_CHUNKS = _COLS_PER_W // _CHUNK_COLS
_GROUPS = _CHUNK_COLS // _L
_PIECE_ROWS = 200
_N_PIECES = DEPTH // _PIECE_ROWS
_N_BUF = 2
_N_TASKS = _N_CHUNKS * _N_PIECES


@functools.partial(
    pl.kernel,
    out_type=jax.ShapeDtypeStruct((DEPTH, BATCH), jnp.float32),
    mesh=plsc.VectorSubcoreMesh(core_axis_name="c", subcore_axis_name="s"),
    compiler_params=pltpu.CompilerParams(
        needs_layout_passes=False, use_tc_tiling_on_sc=True),
    scratch_types=[
        pltpu.VMEM((_COLS_PER_W,), jnp.int32),
        pltpu.VMEM((_PIECE_ROWS, _CHUNK_COLS), jnp.float32),
        pltpu.VMEM((_PIECE_ROWS, _CHUNK_COLS), jnp.float32),
        pltpu.SemaphoreType.DMA,
        pltpu.SemaphoreType.DMA,
    ],
)
def _sc_onehot_t(idx_hbm, out_hbm, idx_v, buf0, buf1, sem0, sem1):
    wid = lax.axis_index("s") * _NC + lax.axis_index("c")
    col0 = wid * _COLS_PER_W

    bufs = (buf0, buf1)
    sems = (sem0, sem1)

    idx_cp = pltpu.make_async_copy(
        idx_hbm.at[pl.ds(col0, _COLS_PER_W)], idx_v, sem0)
    idx_cp.start()

    zero16 = jnp.zeros((_L,), jnp.float32)
    one16 = jnp.full((_L,), 1.0, jnp.float32)
    lanes = lax.iota(jnp.int32, _L)

    def zfill(buf):
        def zbody(r4, carry):
            for dr in range(4):
                for k in range(_GROUPS):
                    buf[r4 * 4 + dr, pl.ds(k * _L, _L)] = zero16
            return carry
        lax.fori_loop(0, _PIECE_ROWS // 4, zbody, 0)

    def piece(task):
        c = task // _N_PIECES
        row0 = pl.multiple_of((task % _N_PIECES) * _PIECE_ROWS, 8)
        return c, row0

    def scatter_piece(buf, task, val16):
        c, row0 = piece(task)
        for g in range(_GROUPS):
            idxv = idx_v[pl.ds(c * _CHUNK_COLS + g * _L, _L)]
            colv = g * _L + lanes
            m = jnp.logical_and(idxv >= row0, idxv < row0 + _PIECE_ROWS)
            plsc.store_scatter(buf, [idxv - row0, colv], val16, mask=m)

    def dma(b, task):
        c, row0 = piece(task)
        cbase = pl.multiple_of(col0 + c * _CHUNK_COLS, _CHUNK_COLS)
        return pltpu.make_async_copy(
            bufs[b],
            out_hbm.at[pl.ds(row0, _PIECE_ROWS), pl.ds(cbase, _CHUNK_COLS)],
            sems[b])

    zfill(buf0)
    idx_cp.wait()
    scatter_piece(buf0, 0, one16)
    dma(0, 0).start()
    zfill(buf1)
    scatter_piece(buf1, 1, one16)
    dma(1, 1).start()

    def lbody(t, carry):
        for j in range(_N_BUF):
            task = _N_BUF * t + j
            dma(j, task - _N_BUF).wait()
            scatter_piece(bufs[j], task - _N_BUF, zero16)
            scatter_piece(bufs[j], task, one16)
            dma(j, task).start()
        return carry

    lax.fori_loop(1, _N_TASKS // _N_BUF, lbody, 0)

    dma(0, _N_TASKS - 2).wait()
    dma(1, _N_TASKS - 1).wait()


@jax.jit
def kernel(X_in, ones):
    del ones
    return _sc_onehot_t(X_in.astype(jnp.int32)).T


# speedup vs baseline: 1.0429x; 1.0429x over previous
import functools

import jax
import jax.numpy as jnp
from jax import lax
from jax.experimental import pallas as pl
from jax.experimental.pallas import tpu as pltpu
from jax.experimental.pallas import tpu_sc as plsc

DEPTH = 1000
BATCH = 16384

_info = plsc.get_sparse_core_info()
_NC, _NS, _L = _info.num_cores, _info.num_subcores, _info.num_lanes
_NW = _NC * _NS
_COLS_PER_W = BATCH // _NW
_CHUNK_COLS = 128
_N_CHUNKS = _COLS_PER_W // _CHUNK_COLS
_GROUPS = _CHUNK_COLS // _L
_PIECE_ROWS = 200
_N_PIECES = DEPTH // _PIECE_ROWS
_N_BUF = 2
_N_TASKS = _N_CHUNKS * _N_PIECES


@functools.partial(
    pl.kernel,
    out_type=jax.ShapeDtypeStruct((DEPTH, BATCH), jnp.float32),
    mesh=plsc.VectorSubcoreMesh(core_axis_name="c", subcore_axis_name="s"),
    compiler_params=pltpu.CompilerParams(
        needs_layout_passes=False, use_tc_tiling_on_sc=True),
    scratch_types=[
        pltpu.VMEM((_COLS_PER_W,), jnp.int32),
        pltpu.VMEM((_PIECE_ROWS, _CHUNK_COLS), jnp.float32),
        pltpu.VMEM((_PIECE_ROWS, _CHUNK_COLS), jnp.float32),
        pltpu.SemaphoreType.DMA,
        pltpu.SemaphoreType.DMA,
    ],
)
def _sc_onehot_t(idx_hbm, out_hbm, idx_v, buf0, buf1, sem0, sem1):
    wid = lax.axis_index("s") * _NC + lax.axis_index("c")
    col0 = wid * _COLS_PER_W

    bufs = (buf0, buf1)
    sems = (sem0, sem1)

    idx_cp = pltpu.make_async_copy(
        idx_hbm.at[pl.ds(col0, _COLS_PER_W)], idx_v, sem0)
    idx_cp.start()

    zero16 = jnp.zeros((_L,), jnp.float32)
    one16 = jnp.full((_L,), 1.0, jnp.float32)
    lanes = lax.iota(jnp.int32, _L)

    def zfill(buf):
        def zbody(r4, carry):
            for dr in range(4):
                for k in range(_GROUPS):
                    buf[r4 * 4 + dr, pl.ds(k * _L, _L)] = zero16
            return carry
        lax.fori_loop(0, _PIECE_ROWS // 4, zbody, 0)

    def piece(task):
        c = task // _N_PIECES
        row0 = pl.multiple_of((task % _N_PIECES) * _PIECE_ROWS, 8)
        return c, row0

    def scatter_piece(buf, task, val16):
        c, row0 = piece(task)
        for g in range(_GROUPS):
            idxv = idx_v[pl.ds(c * _CHUNK_COLS + g * _L, _L)]
            colv = g * _L + lanes
            m = jnp.logical_and(idxv >= row0, idxv < row0 + _PIECE_ROWS)
            plsc.store_scatter(buf, [idxv - row0, colv], val16, mask=m)

    def dma(b, task):
        c, row0 = piece(task)
        cbase = pl.multiple_of(col0 + c * _CHUNK_COLS, _CHUNK_COLS)
        return pltpu.make_async_copy(
            bufs[b],
            out_hbm.at[pl.ds(row0, _PIECE_ROWS), pl.ds(cbase, _CHUNK_COLS)],
            sems[b])

    zfill(buf0)
    idx_cp.wait()
    scatter_piece(buf0, 0, one16)
    dma(0, 0).start()
    zfill(buf1)
    scatter_piece(buf1, 1, one16)
    dma(1, 1).start()

    def lbody(t, carry):
        for j in range(_N_BUF):
            task = _N_BUF * t + j
            dma(j, task - _N_BUF).wait()
            scatter_piece(bufs[j], task - _N_BUF, zero16)
            scatter_piece(bufs[j], task, one16)
            dma(j, task).start()
        return carry

    lax.fori_loop(1, _N_TASKS // _N_BUF, lbody, 0)

    dma(0, _N_TASKS - 2).wait()
    dma(1, _N_TASKS - 1).wait()


@jax.jit
def kernel(X_in, ones):
    del ones
    return _sc_onehot_t(X_in.astype(jnp.int32)).T
